# baseline (speedup 1.0000x reference)
import jax
import jax.numpy as jnp
from jax import lax
from jax.experimental import pallas as pl
from jax.experimental.pallas import tpu as pltpu
from jax.experimental.pallas import tpu_sc as plsc

N = 10000
E = 160000
DF = 128
FH = 150
OUT = 128
HD = 64

NC = 2
NS = 16
LANES = 16

EP = 163840
EC = EP // NC
C = 64
CHUNKS = EC // NS // C
NP = 10240
NA = 10112
FBN = NA // NS

RN = 400
RNP = 512
RNA = 632
REB = 2048


def _tc_node_proj_body(x_ref, wenc_ref, benc_ref, wsrc_ref, wasrc_ref,
                       wadst_ref, h_ref, f_ref, asrc_ref, adst_ref):
  h = jnp.dot(x_ref[...], wenc_ref[...],
              preferred_element_type=jnp.float32) + benc_ref[...]
  h_ref[...] = h
  for out_ref, w_ref in ((f_ref, wsrc_ref), (asrc_ref, wasrc_ref),
                         (adst_ref, wadst_ref)):
    out_ref[...] = jnp.dot(h, w_ref[...], preferred_element_type=jnp.float32)


def _node_proj(x, wencT, benc, wsrcT, wasrcT, wadstT):
  proj = jax.ShapeDtypeStruct((NP, OUT), jnp.float32)
  return pl.pallas_call(
      _tc_node_proj_body,
      grid=(NP // RNP,),
      in_specs=[
          pl.BlockSpec((RNP, DF), lambda i: (i, 0)),
          pl.BlockSpec((DF, FH), lambda i: (0, 0)),
          pl.BlockSpec((1, FH), lambda i: (0, 0)),
          pl.BlockSpec((FH, OUT), lambda i: (0, 0)),
          pl.BlockSpec((FH, OUT), lambda i: (0, 0)),
          pl.BlockSpec((FH, OUT), lambda i: (0, 0)),
      ],
      out_specs=[
          pl.BlockSpec((RNP, FH), lambda i: (i, 0)),
          pl.BlockSpec((RNP, OUT), lambda i: (i, 0)),
          pl.BlockSpec((RNP, OUT), lambda i: (i, 0)),
          pl.BlockSpec((RNP, OUT), lambda i: (i, 0)),
      ],
      out_shape=[
          jax.ShapeDtypeStruct((NP, FH), jnp.float32),
          proj, proj, proj,
      ],
  )(x, wencT, benc, wsrcT, wasrcT, wadstT)


def _tc_edge_attn_body(ea_ref, wee_ref, bee_ref, wae_ref, ae_ref):
  ef = jnp.dot(ea_ref[...], wee_ref[...],
               preferred_element_type=jnp.float32) + bee_ref[...]
  ae_ref[...] = jnp.dot(ef, wae_ref[...], preferred_element_type=jnp.float32)


def _edge_attn(edge_attr, weeT, bee, waeT):
  de = edge_attr.shape[1]
  ee = weeT.shape[1]
  return pl.pallas_call(
      _tc_edge_attn_body,
      grid=(EP // REB,),
      in_specs=[
          pl.BlockSpec((REB, de), lambda i: (i, 0)),
          pl.BlockSpec((de, ee), lambda i: (0, 0)),
          pl.BlockSpec((1, ee), lambda i: (0, 0)),
          pl.BlockSpec((ee, OUT), lambda i: (0, 0)),
      ],
      out_specs=pl.BlockSpec((REB, OUT), lambda i: (i, 0)),
      out_shape=jax.ShapeDtypeStruct((EP, OUT), jnp.float32),
  )(edge_attr, weeT, bee, waeT)


def _tc_merge_body(a_ref, b_ref, f_ref, x_ref, y_ref):
  x_ref[...] = lax.rsqrt(jnp.maximum(a_ref[0] + a_ref[1], 1e-30))
  y_ref[...] = f_ref[...] * lax.rsqrt(
      jnp.maximum(b_ref[0] + b_ref[1], 1e-30))


def _merge(a, b, f):
  out = jax.ShapeDtypeStruct((NA, OUT), jnp.float32)
  return pl.pallas_call(
      _tc_merge_body,
      grid=(NA // RNA,),
      in_specs=[
          pl.BlockSpec((2, RNA, OUT), lambda i: (0, i, 0)),
          pl.BlockSpec((2, RNA, OUT), lambda i: (0, i, 0)),
          pl.BlockSpec((RNA, OUT), lambda i: (i, 0)),
      ],
      out_specs=[
          pl.BlockSpec((RNA, OUT), lambda i: (i, 0)),
          pl.BlockSpec((RNA, OUT), lambda i: (i, 0)),
      ],
      out_shape=[out, out],
  )(a, b, f)


_SC_MESH = plsc.VectorSubcoreMesh(
    core_axis_name="c", subcore_axis_name="s", num_cores=NC, num_subcores=NS)


def _zero_fill(zbuf, sid, acc):
  for q in range(16 * OUT // LANES):
    zbuf[q // (OUT // LANES),
         pl.ds((q % (OUT // LANES)) * LANES, LANES)] = jnp.zeros(
             (LANES,), jnp.float32)
  base_s = sid * FBN
  for k in range(FBN // 16):
    pltpu.sync_copy(zbuf.at[pl.ds(0, 16)],
                    acc.at[pl.ds(base_s + k * 16, 16)])
  pltpu.sync_copy(zbuf.at[pl.ds(0, 8)],
                  acc.at[pl.ds(base_s + (FBN // 16) * 16, 8)])


def _flush(acc, hbm, noff, sid):
  base_s = sid * FBN
  pltpu.sync_copy(acc.at[pl.ds(base_s, FBN)],
                  hbm.at[pl.ds(noff + base_s, FBN)])


def _ebase(cid, sid, j):
  return cid * EC + (j * NS + sid) * C


def _pipeline(fire, work):
  fire(0, 0)

  def pair(p, carry):
    j0 = 2 * p

    @pl.when(j0 + 1 < CHUNKS)
    def _():
      fire(j0 + 1, 1)
    work(j0, 0)

    @pl.when(j0 + 2 < CHUNKS)
    def _():
      fire(j0 + 2, 0)

    @pl.when(j0 + 1 < CHUNKS)
    def _():
      work(j0 + 1, 1)
    return carry

  lax.fori_loop(0, (CHUNKS + 1) // 2, pair, 0)


def _sc_pass_a_body(src_hbm, dst_hbm, asrc_hbm, adst_hbm, ae_hbm,
                    ex_hbm, sdst_hbm,
                    is0, id0, ga0, gb0, ge0, is1, id1, ga1, gb1, ge1,
                    acc, sem0, sem1, semo0, semo1):
  cid = lax.axis_index("c")
  sid = lax.axis_index("s")
  bufs = ((is0, id0, ga0, gb0, ge0, sem0, semo0),
          (is1, id1, ga1, gb1, ge1, sem1, semo1))

  _zero_fill(ga0, sid, acc)
  plsc.subcore_barrier()

  def drain_out(slot):
    _, _, _, _, ge, _, semo = bufs[slot]
    pltpu.make_async_copy(ae_hbm.at[pl.ds(0, C)], ge, semo).wait()

  def fire(j, slot):
    idx_s, idx_d, ga, gb, ge, sem, semo = bufs[slot]

    @pl.when(j >= 2)
    def _():
      drain_out(slot)
    base = _ebase(cid, sid, j)
    pltpu.sync_copy(src_hbm.at[pl.ds(base, C)], idx_s)
    pltpu.sync_copy(dst_hbm.at[pl.ds(base, C)], idx_d)
    pltpu.async_copy(asrc_hbm.at[idx_s], ga, sem)
    pltpu.async_copy(adst_hbm.at[idx_d], gb, sem)
    pltpu.async_copy(ae_hbm.at[pl.ds(base, C)], ge, sem)

  def work(j, slot):
    idx_s, idx_d, ga, gb, ge, sem, semo = bufs[slot]
    base = _ebase(cid, sid, j)
    for b in (ga, gb, ge):
      pltpu.make_async_copy(ae_hbm.at[pl.ds(0, C)], b, sem).wait()

    def row(r, c2):
      for q in range(OUT // LANES):
        sl = pl.ds(q * LANES, LANES)
        g = ga[r, sl] + gb[r, sl] + ge[r, sl]
        g = jnp.maximum(g, 0.2 * g)
        ge[r, sl] = jnp.exp(g)
      return c2

    lax.fori_loop(0, C, row, 0)
    pltpu.async_copy(ge, ex_hbm.at[pl.ds(base, C)], semo)
    pltpu.sync_copy(ge, acc.at[idx_d], add=True)

  _pipeline(fire, work)
  drain_out(0)
  drain_out(1)
  plsc.subcore_barrier()
  _flush(acc, sdst_hbm, cid * NA, sid)


_pass_a = pl.kernel(
    _sc_pass_a_body,
    out_type=[
        jax.ShapeDtypeStruct((EP, OUT), jnp.float32),
        jax.ShapeDtypeStruct((NC * NA, OUT), jnp.float32),
    ],
    mesh=_SC_MESH,
    scratch_types=[
        pltpu.VMEM((C,), jnp.int32),
        pltpu.VMEM((C,), jnp.int32),
        pltpu.VMEM((C, OUT), jnp.float32),
        pltpu.VMEM((C, OUT), jnp.float32),
        pltpu.VMEM((C, OUT), jnp.float32),
        pltpu.VMEM((C,), jnp.int32),
        pltpu.VMEM((C,), jnp.int32),
        pltpu.VMEM((C, OUT), jnp.float32),
        pltpu.VMEM((C, OUT), jnp.float32),
        pltpu.VMEM((C, OUT), jnp.float32),
        pltpu.VMEM_SHARED((NA, OUT), jnp.float32),
        pltpu.SemaphoreType.DMA,
        pltpu.SemaphoreType.DMA,
        pltpu.SemaphoreType.DMA,
        pltpu.SemaphoreType.DMA,
    ],
)


def _sc_pass_a2_body(src_hbm, ex_hbm, ssrc_hbm,
                     is0, ge0, is1, ge1, acc, sem0, sem1):
  cid = lax.axis_index("c")
  sid = lax.axis_index("s")
  bufs = ((is0, ge0, sem0), (is1, ge1, sem1))

  _zero_fill(ge0, sid, acc)
  plsc.subcore_barrier()

  def fire(j, slot):
    idx_s, ge, sem = bufs[slot]
    base = _ebase(cid, sid, j)
    pltpu.sync_copy(src_hbm.at[pl.ds(base, C)], idx_s)
    pltpu.async_copy(ex_hbm.at[pl.ds(base, C)], ge, sem)

  def work(j, slot):
    idx_s, ge, sem = bufs[slot]
    pltpu.make_async_copy(ex_hbm.at[pl.ds(0, C)], ge, sem).wait()
    pltpu.sync_copy(ge, acc.at[idx_s], add=True)

  _pipeline(fire, work)
  plsc.subcore_barrier()
  _flush(acc, ssrc_hbm, cid * NA, sid)


_pass_a2 = pl.kernel(
    _sc_pass_a2_body,
    out_type=jax.ShapeDtypeStruct((NC * NA, OUT), jnp.float32),
    mesh=_SC_MESH,
    scratch_types=[
        pltpu.VMEM((C,), jnp.int32),
        pltpu.VMEM((C, OUT), jnp.float32),
        pltpu.VMEM((C,), jnp.int32),
        pltpu.VMEM((C, OUT), jnp.float32),
        pltpu.VMEM_SHARED((NA, OUT), jnp.float32),
        pltpu.SemaphoreType.DMA,
        pltpu.SemaphoreType.DMA,
    ],
)


def _sc_pass_b_body(src_hbm, dst_hbm, ex_hbm, g_hbm,
                    msg_hbm,
                    is0, id0, bex0, bg0, is1, id1, bex1, bg1,
                    acc, sem0, sem1):
  cid = lax.axis_index("c")
  sid = lax.axis_index("s")
  bufs = ((is0, id0, bex0, bg0, sem0), (is1, id1, bex1, bg1, sem1))

  _zero_fill(bex0, sid, acc)
  plsc.subcore_barrier()

  def fire(j, slot):
    idx_s, idx_d, bex, bg, sem = bufs[slot]
    base = _ebase(cid, sid, j)
    pltpu.sync_copy(src_hbm.at[pl.ds(base, C)], idx_s)
    pltpu.sync_copy(dst_hbm.at[pl.ds(base, C)], idx_d)
    pltpu.async_copy(g_hbm.at[idx_s], bg, sem)
    pltpu.async_copy(ex_hbm.at[pl.ds(base, C)], bex, sem)

  def work(j, slot):
    idx_s, idx_d, bex, bg, sem = bufs[slot]
    for b in (bex, bg):
      pltpu.make_async_copy(ex_hbm.at[pl.ds(0, C)], b, sem).wait()

    def row(r, c2):
      for q in range(OUT // LANES):
        sl = pl.ds(q * LANES, LANES)
        bg[r, sl] = bg[r, sl] * bex[r, sl]
      return c2

    lax.fori_loop(0, C, row, 0)
    pltpu.sync_copy(bg, acc.at[idx_d], add=True)

  _pipeline(fire, work)
  plsc.subcore_barrier()
  _flush(acc, msg_hbm, cid * NA, sid)


_pass_b = pl.kernel(
    _sc_pass_b_body,
    out_type=jax.ShapeDtypeStruct((NC * NA, OUT), jnp.float32),
    mesh=_SC_MESH,
    scratch_types=[
        pltpu.VMEM((C,), jnp.int32),
        pltpu.VMEM((C,), jnp.int32),
        pltpu.VMEM((C, OUT), jnp.float32),
        pltpu.VMEM((C, OUT), jnp.float32),
        pltpu.VMEM((C,), jnp.int32),
        pltpu.VMEM((C,), jnp.int32),
        pltpu.VMEM((C, OUT), jnp.float32),
        pltpu.VMEM((C, OUT), jnp.float32),
        pltpu.VMEM_SHARED((NA, OUT), jnp.float32),
        pltpu.SemaphoreType.DMA,
        pltpu.SemaphoreType.DMA,
    ],
)


def _tc_final_body(msg_ref, rsd_ref, h_ref, scl_ref, off_ref, waggT_ref,
                   bagg_ref, wdstT_ref, bdst_ref, out_ref):
  acc = bagg_ref[...] + bdst_ref[...] + jnp.dot(
      h_ref[...], wdstT_ref[...], preferred_element_type=jnp.float32)
  msg = (msg_ref[0] + msg_ref[1]) * rsd_ref[...]
  waggT = waggT_ref[...]
  for hh in range(2):
    m = msg[:, hh * HD:(hh + 1) * HD]
    mean = jnp.mean(m, axis=1, keepdims=True)
    d = m - mean
    var = jnp.mean(d * d, axis=1, keepdims=True)
    hn = d * scl_ref[0, hh][None, :] * lax.rsqrt(var + 1e-9) \
        + off_ref[0, hh][None, :]
    acc = acc + jnp.dot(hn, waggT[hh * HD:(hh + 1) * HD, :],
                        preferred_element_type=jnp.float32)
  out_ref[...] = acc


def _final(msg, rsd, h, scale, offset, waggT, bagg, wdstT, bdst):
  return pl.pallas_call(
      _tc_final_body,
      grid=(N // RN,),
      in_specs=[
          pl.BlockSpec((2, RN, OUT), lambda i: (0, i, 0)),
          pl.BlockSpec((RN, OUT), lambda i: (i, 0)),
          pl.BlockSpec((RN, FH), lambda i: (i, 0)),
          pl.BlockSpec((1, 2, HD), lambda i: (0, 0, 0)),
          pl.BlockSpec((1, 2, HD), lambda i: (0, 0, 0)),
          pl.BlockSpec((OUT, OUT), lambda i: (0, 0)),
          pl.BlockSpec((1, OUT), lambda i: (0, 0)),
          pl.BlockSpec((FH, OUT), lambda i: (0, 0)),
          pl.BlockSpec((1, OUT), lambda i: (0, 0)),
      ],
      out_specs=pl.BlockSpec((RN, OUT), lambda i: (i, 0)),
      out_shape=jax.ShapeDtypeStruct((N, OUT), jnp.float32),
  )(msg, rsd, h, scale, offset, waggT, bagg, wdstT, bdst)


def kernel(x, edge_index, edge_attr, W_enc, b_enc, W_ee, b_ee, W_src, W_asrc,
           W_adst, W_aedge, scale, offset, W_agg, b_agg, W_dst, b_dst):
  src = jnp.concatenate(
      [edge_index[0].astype(jnp.int32),
       jnp.full((EP - E,), N, dtype=jnp.int32)])
  dst = jnp.concatenate(
      [edge_index[1].astype(jnp.int32),
       jnp.full((EP - E,), N, dtype=jnp.int32)])
  xp = jnp.zeros((NP, DF), jnp.float32).at[:N].set(x)
  eap = jnp.zeros((EP, edge_attr.shape[1]), jnp.float32).at[:E].set(edge_attr)

  h, f, asrc, adst = _node_proj(xp, W_enc.T, b_enc[None, :], W_src.T,
                                W_asrc.T, W_adst.T)
  ae = _edge_attn(eap, W_ee.T, b_ee[None, :], W_aedge.T)

  ex, sdst_p = _pass_a(src, dst, asrc, adst, ae)
  ssrc_p = _pass_a2(src, ex)
  rsd, g = _merge(sdst_p.reshape(2, NA, OUT), ssrc_p.reshape(2, NA, OUT), f)
  msg_p = _pass_b(src, dst, ex, g)

  return _final(msg_p.reshape(2, NA, OUT), rsd, h, scale, offset, W_agg.T,
                b_agg[None, :], W_dst.T, b_dst[None, :])


# speedup vs baseline: 1.4125x; 1.4125x over previous
import jax
import jax.numpy as jnp
from jax import lax
from jax.experimental import pallas as pl
from jax.experimental.pallas import tpu as pltpu
from jax.experimental.pallas import tpu_sc as plsc

N = 10000
E = 160000
DF = 128
FH = 150
OUT = 128
HD = 64

NC = 2
NS = 16
LANES = 16

EP = 163840
EC = EP // NC
C = 64
CHUNKS = EC // NS // C
NP = 10240
NA = 10112
FBN = NA // NS

RN = 400
RNP = 512
RNA = 632
REB = 2048


def _tc_node_proj_body(x_ref, wenc_ref, benc_ref, wsrc_ref, wasrc_ref,
                       wadst_ref, h_ref, f_ref, asrc_ref, adst_ref):
  h = jnp.dot(x_ref[...], wenc_ref[...],
              preferred_element_type=jnp.float32) + benc_ref[...]
  h_ref[...] = h
  for out_ref, w_ref in ((f_ref, wsrc_ref), (asrc_ref, wasrc_ref),
                         (adst_ref, wadst_ref)):
    out_ref[...] = jnp.dot(h, w_ref[...], preferred_element_type=jnp.float32)


def _node_proj(x, wencT, benc, wsrcT, wasrcT, wadstT):
  proj = jax.ShapeDtypeStruct((NP, OUT), jnp.float32)
  return pl.pallas_call(
      _tc_node_proj_body,
      grid=(NP // RNP,),
      in_specs=[
          pl.BlockSpec((RNP, DF), lambda i: (i, 0)),
          pl.BlockSpec((DF, FH), lambda i: (0, 0)),
          pl.BlockSpec((1, FH), lambda i: (0, 0)),
          pl.BlockSpec((FH, OUT), lambda i: (0, 0)),
          pl.BlockSpec((FH, OUT), lambda i: (0, 0)),
          pl.BlockSpec((FH, OUT), lambda i: (0, 0)),
      ],
      out_specs=[
          pl.BlockSpec((RNP, FH), lambda i: (i, 0)),
          pl.BlockSpec((RNP, OUT), lambda i: (i, 0)),
          pl.BlockSpec((RNP, OUT), lambda i: (i, 0)),
          pl.BlockSpec((RNP, OUT), lambda i: (i, 0)),
      ],
      out_shape=[
          jax.ShapeDtypeStruct((NP, FH), jnp.float32),
          proj, proj, proj,
      ],
  )(x, wencT, benc, wsrcT, wasrcT, wadstT)


def _tc_edge_attn_body(ea_ref, wee_ref, bee_ref, wae_ref, ae_ref):
  ef = jnp.dot(ea_ref[...], wee_ref[...],
               preferred_element_type=jnp.float32) + bee_ref[...]
  ae_ref[...] = jnp.dot(ef, wae_ref[...], preferred_element_type=jnp.float32)


def _edge_attn(edge_attr, weeT, bee, waeT):
  de = edge_attr.shape[1]
  ee = weeT.shape[1]
  return pl.pallas_call(
      _tc_edge_attn_body,
      grid=(EP // REB,),
      in_specs=[
          pl.BlockSpec((REB, de), lambda i: (i, 0)),
          pl.BlockSpec((de, ee), lambda i: (0, 0)),
          pl.BlockSpec((1, ee), lambda i: (0, 0)),
          pl.BlockSpec((ee, OUT), lambda i: (0, 0)),
      ],
      out_specs=pl.BlockSpec((REB, OUT), lambda i: (i, 0)),
      out_shape=jax.ShapeDtypeStruct((EP, OUT), jnp.float32),
  )(edge_attr, weeT, bee, waeT)


def _tc_merge_body(a_ref, b_ref, f_ref, x_ref, y_ref):
  x_ref[...] = lax.rsqrt(jnp.maximum(a_ref[0] + a_ref[1], 1e-30))
  y_ref[...] = f_ref[...] * lax.rsqrt(
      jnp.maximum(b_ref[0] + b_ref[1], 1e-30))


def _merge(a, b, f):
  out = jax.ShapeDtypeStruct((NA, OUT), jnp.float32)
  return pl.pallas_call(
      _tc_merge_body,
      grid=(NA // RNA,),
      in_specs=[
          pl.BlockSpec((2, RNA, OUT), lambda i: (0, i, 0)),
          pl.BlockSpec((2, RNA, OUT), lambda i: (0, i, 0)),
          pl.BlockSpec((RNA, OUT), lambda i: (i, 0)),
      ],
      out_specs=[
          pl.BlockSpec((RNA, OUT), lambda i: (i, 0)),
          pl.BlockSpec((RNA, OUT), lambda i: (i, 0)),
      ],
      out_shape=[out, out],
  )(a, b, f)


_SC_MESH = plsc.VectorSubcoreMesh(
    core_axis_name="c", subcore_axis_name="s", num_cores=NC, num_subcores=NS)


def _zero_fill(zbuf, sid, acc):
  for q in range(16 * OUT // LANES):
    zbuf[q // (OUT // LANES),
         pl.ds((q % (OUT // LANES)) * LANES, LANES)] = jnp.zeros(
             (LANES,), jnp.float32)
  base_s = sid * FBN
  for k in range(FBN // 16):
    pltpu.sync_copy(zbuf.at[pl.ds(0, 16)],
                    acc.at[pl.ds(base_s + k * 16, 16)])
  pltpu.sync_copy(zbuf.at[pl.ds(0, 8)],
                  acc.at[pl.ds(base_s + (FBN // 16) * 16, 8)])


def _flush(acc, hbm, noff, sid):
  base_s = sid * FBN
  pltpu.sync_copy(acc.at[pl.ds(base_s, FBN)],
                  hbm.at[pl.ds(noff + base_s, FBN)])


def _ebase(cid, sid, j):
  return cid * EC + (j * NS + sid) * C


def _pipeline(fire, work):
  fire(0, 0)

  def pair(p, carry):
    j0 = 2 * p

    @pl.when(j0 + 1 < CHUNKS)
    def _():
      fire(j0 + 1, 1)
    work(j0, 0)

    @pl.when(j0 + 2 < CHUNKS)
    def _():
      fire(j0 + 2, 0)

    @pl.when(j0 + 1 < CHUNKS)
    def _():
      work(j0 + 1, 1)
    return carry

  lax.fori_loop(0, (CHUNKS + 1) // 2, pair, 0)


def _sc_pass_a_body(src_hbm, dst_hbm, asrc_hbm, adst_hbm, ae_hbm,
                    ex_hbm, sdst_hbm,
                    is0, id0, ga0, gb0, ge0, is1, id1, ga1, gb1, ge1,
                    acc, sem0, sem1, semo0, semo1):
  cid = lax.axis_index("c")
  sid = lax.axis_index("s")
  bufs = ((is0, id0, ga0, gb0, ge0, sem0, semo0),
          (is1, id1, ga1, gb1, ge1, sem1, semo1))

  _zero_fill(ga0, sid, acc)
  plsc.subcore_barrier()

  def drain_out(slot):
    _, _, _, _, ge, _, semo = bufs[slot]
    pltpu.make_async_copy(ae_hbm.at[pl.ds(0, C)], ge, semo).wait()

  def fire(j, slot):
    idx_s, idx_d, ga, gb, ge, sem, semo = bufs[slot]

    @pl.when(j >= 2)
    def _():
      drain_out(slot)
    base = _ebase(cid, sid, j)
    pltpu.sync_copy(src_hbm.at[pl.ds(base, C)], idx_s)
    pltpu.sync_copy(dst_hbm.at[pl.ds(base, C)], idx_d)
    pltpu.async_copy(asrc_hbm.at[idx_s], ga, sem)
    pltpu.async_copy(adst_hbm.at[idx_d], gb, sem)
    pltpu.async_copy(ae_hbm.at[pl.ds(base, C)], ge, sem)

  def work(j, slot):
    idx_s, idx_d, ga, gb, ge, sem, semo = bufs[slot]
    base = _ebase(cid, sid, j)
    for b in (ga, gb, ge):
      pltpu.make_async_copy(ae_hbm.at[pl.ds(0, C)], b, sem).wait()

    def row(r, c2):
      for q in range(OUT // LANES):
        sl = pl.ds(q * LANES, LANES)
        g = ga[r, sl] + gb[r, sl] + ge[r, sl]
        g = jnp.maximum(g, 0.2 * g)
        ge[r, sl] = jnp.exp(g)
      return c2

    lax.fori_loop(0, C, row, 0)
    pltpu.async_copy(ge, ex_hbm.at[pl.ds(base, C)], semo)
    pltpu.sync_copy(ge, acc.at[idx_d], add=True)

  _pipeline(fire, work)
  drain_out(0)
  drain_out(1)
  plsc.subcore_barrier()
  _flush(acc, sdst_hbm, cid * NA, sid)


_pass_a = pl.kernel(
    _sc_pass_a_body,
    out_type=[
        jax.ShapeDtypeStruct((EP, OUT), jnp.float32),
        jax.ShapeDtypeStruct((NC * NA, OUT), jnp.float32),
    ],
    mesh=_SC_MESH,
    scratch_types=[
        pltpu.VMEM((C,), jnp.int32),
        pltpu.VMEM((C,), jnp.int32),
        pltpu.VMEM((C, OUT), jnp.float32),
        pltpu.VMEM((C, OUT), jnp.float32),
        pltpu.VMEM((C, OUT), jnp.float32),
        pltpu.VMEM((C,), jnp.int32),
        pltpu.VMEM((C,), jnp.int32),
        pltpu.VMEM((C, OUT), jnp.float32),
        pltpu.VMEM((C, OUT), jnp.float32),
        pltpu.VMEM((C, OUT), jnp.float32),
        pltpu.VMEM_SHARED((NA, OUT), jnp.float32),
        pltpu.SemaphoreType.DMA,
        pltpu.SemaphoreType.DMA,
        pltpu.SemaphoreType.DMA,
        pltpu.SemaphoreType.DMA,
    ],
)


def _sc_pass_a2_body(src_hbm, ex_hbm, ssrc_hbm,
                     is0, ge0, is1, ge1, acc, sem0, sem1):
  cid = lax.axis_index("c")
  sid = lax.axis_index("s")
  bufs = ((is0, ge0, sem0), (is1, ge1, sem1))

  _zero_fill(ge0, sid, acc)
  plsc.subcore_barrier()

  def fire(j, slot):
    idx_s, ge, sem = bufs[slot]
    base = _ebase(cid, sid, j)
    pltpu.sync_copy(src_hbm.at[pl.ds(base, C)], idx_s)
    pltpu.async_copy(ex_hbm.at[pl.ds(base, C)], ge, sem)

  def work(j, slot):
    idx_s, ge, sem = bufs[slot]
    pltpu.make_async_copy(ex_hbm.at[pl.ds(0, C)], ge, sem).wait()
    pltpu.sync_copy(ge, acc.at[idx_s], add=True)

  _pipeline(fire, work)
  plsc.subcore_barrier()
  _flush(acc, ssrc_hbm, cid * NA, sid)


_pass_a2 = pl.kernel(
    _sc_pass_a2_body,
    out_type=jax.ShapeDtypeStruct((NC * NA, OUT), jnp.float32),
    mesh=_SC_MESH,
    scratch_types=[
        pltpu.VMEM((C,), jnp.int32),
        pltpu.VMEM((C, OUT), jnp.float32),
        pltpu.VMEM((C,), jnp.int32),
        pltpu.VMEM((C, OUT), jnp.float32),
        pltpu.VMEM_SHARED((NA, OUT), jnp.float32),
        pltpu.SemaphoreType.DMA,
        pltpu.SemaphoreType.DMA,
    ],
)


def _sc_pass_b_body(src_hbm, dst_hbm, ex_hbm, g_hbm,
                    msg_hbm,
                    is0, id0, bex0, bg0, is1, id1, bex1, bg1,
                    acc, sem0, sem1):
  cid = lax.axis_index("c")
  sid = lax.axis_index("s")
  bufs = ((is0, id0, bex0, bg0, sem0), (is1, id1, bex1, bg1, sem1))

  _zero_fill(bex0, sid, acc)
  plsc.subcore_barrier()

  def fire(j, slot):
    idx_s, idx_d, bex, bg, sem = bufs[slot]
    base = _ebase(cid, sid, j)
    pltpu.sync_copy(src_hbm.at[pl.ds(base, C)], idx_s)
    pltpu.sync_copy(dst_hbm.at[pl.ds(base, C)], idx_d)
    pltpu.async_copy(g_hbm.at[idx_s], bg, sem)
    pltpu.async_copy(ex_hbm.at[pl.ds(base, C)], bex, sem)

  def work(j, slot):
    idx_s, idx_d, bex, bg, sem = bufs[slot]
    for b in (bex, bg):
      pltpu.make_async_copy(ex_hbm.at[pl.ds(0, C)], b, sem).wait()

    def row(r, c2):
      for q in range(OUT // LANES):
        sl = pl.ds(q * LANES, LANES)
        bg[r, sl] = bg[r, sl] * bex[r, sl]
      return c2

    lax.fori_loop(0, C, row, 0)
    pltpu.sync_copy(bg, acc.at[idx_d], add=True)

  _pipeline(fire, work)
  plsc.subcore_barrier()
  _flush(acc, msg_hbm, cid * NA, sid)


_pass_b = pl.kernel(
    _sc_pass_b_body,
    out_type=jax.ShapeDtypeStruct((NC * NA, OUT), jnp.float32),
    mesh=_SC_MESH,
    scratch_types=[
        pltpu.VMEM((C,), jnp.int32),
        pltpu.VMEM((C,), jnp.int32),
        pltpu.VMEM((C, OUT), jnp.float32),
        pltpu.VMEM((C, OUT), jnp.float32),
        pltpu.VMEM((C,), jnp.int32),
        pltpu.VMEM((C,), jnp.int32),
        pltpu.VMEM((C, OUT), jnp.float32),
        pltpu.VMEM((C, OUT), jnp.float32),
        pltpu.VMEM_SHARED((NA, OUT), jnp.float32),
        pltpu.SemaphoreType.DMA,
        pltpu.SemaphoreType.DMA,
    ],
)


def _tc_final_body(msg_ref, rsd_ref, h_ref, scl_ref, off_ref, waggT_ref,
                   bagg_ref, wdstT_ref, bdst_ref, out_ref):
  acc = bagg_ref[...] + bdst_ref[...] + jnp.dot(
      h_ref[...], wdstT_ref[...], preferred_element_type=jnp.float32)
  msg = (msg_ref[0] + msg_ref[1]) * rsd_ref[...]
  waggT = waggT_ref[...]
  for hh in range(2):
    m = msg[:, hh * HD:(hh + 1) * HD]
    mean = jnp.mean(m, axis=1, keepdims=True)
    d = m - mean
    var = jnp.mean(d * d, axis=1, keepdims=True)
    hn = d * scl_ref[0, hh][None, :] * lax.rsqrt(var + 1e-9) \
        + off_ref[0, hh][None, :]
    acc = acc + jnp.dot(hn, waggT[hh * HD:(hh + 1) * HD, :],
                        preferred_element_type=jnp.float32)
  out_ref[...] = acc


def _final(msg, rsd, h, scale, offset, waggT, bagg, wdstT, bdst):
  return pl.pallas_call(
      _tc_final_body,
      grid=(N // RN,),
      in_specs=[
          pl.BlockSpec((2, RN, OUT), lambda i: (0, i, 0)),
          pl.BlockSpec((RN, OUT), lambda i: (i, 0)),
          pl.BlockSpec((RN, FH), lambda i: (i, 0)),
          pl.BlockSpec((1, 2, HD), lambda i: (0, 0, 0)),
          pl.BlockSpec((1, 2, HD), lambda i: (0, 0, 0)),
          pl.BlockSpec((OUT, OUT), lambda i: (0, 0)),
          pl.BlockSpec((1, OUT), lambda i: (0, 0)),
          pl.BlockSpec((FH, OUT), lambda i: (0, 0)),
          pl.BlockSpec((1, OUT), lambda i: (0, 0)),
      ],
      out_specs=pl.BlockSpec((RN, OUT), lambda i: (i, 0)),
      out_shape=jax.ShapeDtypeStruct((N, OUT), jnp.float32),
  )(msg, rsd, h, scale, offset, waggT, bagg, wdstT, bdst)


def kernel(x, edge_index, edge_attr, W_enc, b_enc, W_ee, b_ee, W_src, W_asrc,
           W_adst, W_aedge, scale, offset, W_agg, b_agg, W_dst, b_dst):
  pad_idx = N + jnp.arange(EP - E, dtype=jnp.int32) % (NA - N)
  src = jnp.concatenate([edge_index[0].astype(jnp.int32), pad_idx])
  dst = jnp.concatenate([edge_index[1].astype(jnp.int32), pad_idx])
  xp = jnp.zeros((NP, DF), jnp.float32).at[:N].set(x)
  eap = jnp.zeros((EP, edge_attr.shape[1]), jnp.float32).at[:E].set(edge_attr)

  h, f, asrc, adst = _node_proj(xp, W_enc.T, b_enc[None, :], W_src.T,
                                W_asrc.T, W_adst.T)
  ae = _edge_attn(eap, W_ee.T, b_ee[None, :], W_aedge.T)

  ex, sdst_p = _pass_a(src, dst, asrc, adst, ae)
  ssrc_p = _pass_a2(src, ex)
  rsd, g = _merge(sdst_p.reshape(2, NA, OUT), ssrc_p.reshape(2, NA, OUT), f)
  msg_p = _pass_b(src, dst, ex, g)

  return _final(msg_p.reshape(2, NA, OUT), rsd, h, scale, offset, W_agg.T,
                b_agg[None, :], W_dst.T, b_dst[None, :])


# speedup vs baseline: 1.4524x; 1.0283x over previous
import jax
import jax.numpy as jnp
from jax import lax
from jax.experimental import pallas as pl
from jax.experimental.pallas import tpu as pltpu
from jax.experimental.pallas import tpu_sc as plsc

N = 10000
E = 160000
DF = 128
FH = 150
OUT = 128
HD = 64

NC = 2
NS = 16
LANES = 16

EP = 163840
EC = EP // NC
C = 64
CHUNKS = EC // NS // C
NP = 10240
NA = 10112
FBN = NA // NS

RN = 400
RNP = 512
RNA = 632
REB = 2048


def _tc_node_proj_body(x_ref, wenc_ref, benc_ref, wsrc_ref, wasrc_ref,
                       wadst_ref, h_ref, f_ref, asrc_ref, adst_ref):
  h = jnp.dot(x_ref[...], wenc_ref[...],
              preferred_element_type=jnp.float32) + benc_ref[...]
  h_ref[...] = h
  for out_ref, w_ref in ((f_ref, wsrc_ref), (asrc_ref, wasrc_ref),
                         (adst_ref, wadst_ref)):
    out_ref[...] = jnp.dot(h, w_ref[...], preferred_element_type=jnp.float32)


def _node_proj(x, wencT, benc, wsrcT, wasrcT, wadstT):
  proj = jax.ShapeDtypeStruct((NP, OUT), jnp.float32)
  return pl.pallas_call(
      _tc_node_proj_body,
      grid=(NP // RNP,),
      in_specs=[
          pl.BlockSpec((RNP, DF), lambda i: (i, 0)),
          pl.BlockSpec((DF, FH), lambda i: (0, 0)),
          pl.BlockSpec((1, FH), lambda i: (0, 0)),
          pl.BlockSpec((FH, OUT), lambda i: (0, 0)),
          pl.BlockSpec((FH, OUT), lambda i: (0, 0)),
          pl.BlockSpec((FH, OUT), lambda i: (0, 0)),
      ],
      out_specs=[
          pl.BlockSpec((RNP, FH), lambda i: (i, 0)),
          pl.BlockSpec((RNP, OUT), lambda i: (i, 0)),
          pl.BlockSpec((RNP, OUT), lambda i: (i, 0)),
          pl.BlockSpec((RNP, OUT), lambda i: (i, 0)),
      ],
      out_shape=[
          jax.ShapeDtypeStruct((NP, FH), jnp.float32),
          proj, proj, proj,
      ],
  )(x, wencT, benc, wsrcT, wasrcT, wadstT)


def _tc_edge_attn_body(ea_ref, wee_ref, bee_ref, wae_ref, ae_ref):
  ef = jnp.dot(ea_ref[...], wee_ref[...],
               preferred_element_type=jnp.float32) + bee_ref[...]
  ae_ref[...] = jnp.dot(ef, wae_ref[...], preferred_element_type=jnp.float32)


def _edge_attn(edge_attr, weeT, bee, waeT):
  de = edge_attr.shape[1]
  ee = weeT.shape[1]
  return pl.pallas_call(
      _tc_edge_attn_body,
      grid=(EP // REB,),
      in_specs=[
          pl.BlockSpec((REB, de), lambda i: (i, 0)),
          pl.BlockSpec((de, ee), lambda i: (0, 0)),
          pl.BlockSpec((1, ee), lambda i: (0, 0)),
          pl.BlockSpec((ee, OUT), lambda i: (0, 0)),
      ],
      out_specs=pl.BlockSpec((REB, OUT), lambda i: (i, 0)),
      out_shape=jax.ShapeDtypeStruct((EP, OUT), jnp.float32),
  )(edge_attr, weeT, bee, waeT)


def _tc_merge_body(a_ref, b_ref, f_ref, x_ref, y_ref):
  x_ref[...] = lax.rsqrt(jnp.maximum(a_ref[0] + a_ref[1], 1e-30))
  y_ref[...] = f_ref[...] * lax.rsqrt(
      jnp.maximum(b_ref[0] + b_ref[1], 1e-30))


def _merge(a, b, f):
  out = jax.ShapeDtypeStruct((NA, OUT), jnp.float32)
  return pl.pallas_call(
      _tc_merge_body,
      grid=(NA // RNA,),
      in_specs=[
          pl.BlockSpec((2, RNA, OUT), lambda i: (0, i, 0)),
          pl.BlockSpec((2, RNA, OUT), lambda i: (0, i, 0)),
          pl.BlockSpec((RNA, OUT), lambda i: (i, 0)),
      ],
      out_specs=[
          pl.BlockSpec((RNA, OUT), lambda i: (i, 0)),
          pl.BlockSpec((RNA, OUT), lambda i: (i, 0)),
      ],
      out_shape=[out, out],
  )(a, b, f)


_SC_MESH = plsc.VectorSubcoreMesh(
    core_axis_name="c", subcore_axis_name="s", num_cores=NC, num_subcores=NS)


def _zero_fill(zbuf, sid, acc):
  for q in range(16 * OUT // LANES):
    zbuf[q // (OUT // LANES),
         pl.ds((q % (OUT // LANES)) * LANES, LANES)] = jnp.zeros(
             (LANES,), jnp.float32)
  base_s = sid * FBN
  for k in range(FBN // 16):
    pltpu.sync_copy(zbuf.at[pl.ds(0, 16)],
                    acc.at[pl.ds(base_s + k * 16, 16)])
  pltpu.sync_copy(zbuf.at[pl.ds(0, 8)],
                  acc.at[pl.ds(base_s + (FBN // 16) * 16, 8)])


def _flush(acc, hbm, noff, sid):
  base_s = sid * FBN
  pltpu.sync_copy(acc.at[pl.ds(base_s, FBN)],
                  hbm.at[pl.ds(noff + base_s, FBN)])


def _ebase(cid, sid, j):
  return cid * EC + (j * NS + sid) * C


def _pipeline(fire, work, chunks=CHUNKS):
  fire(0, 0)

  def pair(p, carry):
    j0 = 2 * p

    @pl.when(j0 + 1 < chunks)
    def _():
      fire(j0 + 1, 1)
    work(j0, 0)

    @pl.when(j0 + 2 < chunks)
    def _():
      fire(j0 + 2, 0)

    @pl.when(j0 + 1 < chunks)
    def _():
      work(j0 + 1, 1)
    return carry

  lax.fori_loop(0, (chunks + 1) // 2, pair, 0)


def _sc_pass_a_body(src_hbm, dst_hbm, asrc_hbm, adst_hbm, ae_hbm,
                    ex_hbm, sdst_hbm,
                    is0, id0, ga0, gb0, ge0, is1, id1, ga1, gb1, ge1,
                    acc, sem0, sem1, semo0, semo1):
  cid = lax.axis_index("c")
  sid = lax.axis_index("s")
  bufs = ((is0, id0, ga0, gb0, ge0, sem0, semo0),
          (is1, id1, ga1, gb1, ge1, sem1, semo1))

  _zero_fill(ga0, sid, acc)
  plsc.subcore_barrier()

  def drain_out(slot):
    _, _, _, _, ge, _, semo = bufs[slot]
    pltpu.make_async_copy(ae_hbm.at[pl.ds(0, C)], ge, semo).wait()

  def fire(j, slot):
    idx_s, idx_d, ga, gb, ge, sem, semo = bufs[slot]

    @pl.when(j >= 2)
    def _():
      drain_out(slot)
    base = _ebase(cid, sid, j)
    pltpu.sync_copy(src_hbm.at[pl.ds(base, C)], idx_s)
    pltpu.sync_copy(dst_hbm.at[pl.ds(base, C)], idx_d)
    pltpu.async_copy(asrc_hbm.at[idx_s], ga, sem)
    pltpu.async_copy(adst_hbm.at[idx_d], gb, sem)
    pltpu.async_copy(ae_hbm.at[pl.ds(base, C)], ge, sem)

  def work(j, slot):
    idx_s, idx_d, ga, gb, ge, sem, semo = bufs[slot]
    base = _ebase(cid, sid, j)
    for b in (ga, gb, ge):
      pltpu.make_async_copy(ae_hbm.at[pl.ds(0, C)], b, sem).wait()

    def row(r, c2):
      for q in range(OUT // LANES):
        sl = pl.ds(q * LANES, LANES)
        g = ga[r, sl] + gb[r, sl] + ge[r, sl]
        g = jnp.maximum(g, 0.2 * g)
        ge[r, sl] = jnp.exp(g)
      return c2

    lax.fori_loop(0, C, row, 0)
    pltpu.async_copy(ge, ex_hbm.at[pl.ds(base, C)], semo)
    pltpu.sync_copy(ge, acc.at[idx_d], add=True)

  _pipeline(fire, work)
  drain_out(0)
  drain_out(1)
  plsc.subcore_barrier()
  _flush(acc, sdst_hbm, cid * NA, sid)


_pass_a = pl.kernel(
    _sc_pass_a_body,
    out_type=[
        jax.ShapeDtypeStruct((EP, OUT), jnp.float32),
        jax.ShapeDtypeStruct((NC * NA, OUT), jnp.float32),
    ],
    mesh=_SC_MESH,
    scratch_types=[
        pltpu.VMEM((C,), jnp.int32),
        pltpu.VMEM((C,), jnp.int32),
        pltpu.VMEM((C, OUT), jnp.float32),
        pltpu.VMEM((C, OUT), jnp.float32),
        pltpu.VMEM((C, OUT), jnp.float32),
        pltpu.VMEM((C,), jnp.int32),
        pltpu.VMEM((C,), jnp.int32),
        pltpu.VMEM((C, OUT), jnp.float32),
        pltpu.VMEM((C, OUT), jnp.float32),
        pltpu.VMEM((C, OUT), jnp.float32),
        pltpu.VMEM_SHARED((NA, OUT), jnp.float32),
        pltpu.SemaphoreType.DMA,
        pltpu.SemaphoreType.DMA,
        pltpu.SemaphoreType.DMA,
        pltpu.SemaphoreType.DMA,
    ],
)


C2 = 128
CHUNKS2 = EC // NS // C2


def _sc_pass_a2_body(src_hbm, ex_hbm, ssrc_hbm,
                     is0, ge0, is1, ge1, acc, sem0, sem1):
  cid = lax.axis_index("c")
  sid = lax.axis_index("s")
  bufs = ((is0, ge0, sem0), (is1, ge1, sem1))

  _zero_fill(ge0, sid, acc)
  plsc.subcore_barrier()

  def fire(j, slot):
    idx_s, ge, sem = bufs[slot]
    base = cid * EC + (j * NS + sid) * C2
    pltpu.sync_copy(src_hbm.at[pl.ds(base, C2)], idx_s)
    pltpu.async_copy(ex_hbm.at[pl.ds(base, C2)], ge, sem)

  def work(j, slot):
    idx_s, ge, sem = bufs[slot]
    pltpu.make_async_copy(ex_hbm.at[pl.ds(0, C2)], ge, sem).wait()
    pltpu.sync_copy(ge, acc.at[idx_s], add=True)

  _pipeline(fire, work, CHUNKS2)
  plsc.subcore_barrier()
  _flush(acc, ssrc_hbm, cid * NA, sid)


_pass_a2 = pl.kernel(
    _sc_pass_a2_body,
    out_type=jax.ShapeDtypeStruct((NC * NA, OUT), jnp.float32),
    mesh=_SC_MESH,
    scratch_types=[
        pltpu.VMEM((C2,), jnp.int32),
        pltpu.VMEM((C2, OUT), jnp.float32),
        pltpu.VMEM((C2,), jnp.int32),
        pltpu.VMEM((C2, OUT), jnp.float32),
        pltpu.VMEM_SHARED((NA, OUT), jnp.float32),
        pltpu.SemaphoreType.DMA,
        pltpu.SemaphoreType.DMA,
    ],
)


def _sc_pass_b_body(src_hbm, dst_hbm, ex_hbm, g_hbm,
                    msg_hbm,
                    is0, id0, bex0, bg0, is1, id1, bex1, bg1,
                    acc, sem0, sem1):
  cid = lax.axis_index("c")
  sid = lax.axis_index("s")
  bufs = ((is0, id0, bex0, bg0, sem0), (is1, id1, bex1, bg1, sem1))

  _zero_fill(bex0, sid, acc)
  plsc.subcore_barrier()

  def fire(j, slot):
    idx_s, idx_d, bex, bg, sem = bufs[slot]
    base = _ebase(cid, sid, j)
    pltpu.sync_copy(src_hbm.at[pl.ds(base, C)], idx_s)
    pltpu.sync_copy(dst_hbm.at[pl.ds(base, C)], idx_d)
    pltpu.async_copy(g_hbm.at[idx_s], bg, sem)
    pltpu.async_copy(ex_hbm.at[pl.ds(base, C)], bex, sem)

  def work(j, slot):
    idx_s, idx_d, bex, bg, sem = bufs[slot]
    for b in (bex, bg):
      pltpu.make_async_copy(ex_hbm.at[pl.ds(0, C)], b, sem).wait()

    def row(r, c2):
      for q in range(OUT // LANES):
        sl = pl.ds(q * LANES, LANES)
        bg[r, sl] = bg[r, sl] * bex[r, sl]
      return c2

    lax.fori_loop(0, C, row, 0)
    pltpu.sync_copy(bg, acc.at[idx_d], add=True)

  _pipeline(fire, work)
  plsc.subcore_barrier()
  _flush(acc, msg_hbm, cid * NA, sid)


_pass_b = pl.kernel(
    _sc_pass_b_body,
    out_type=jax.ShapeDtypeStruct((NC * NA, OUT), jnp.float32),
    mesh=_SC_MESH,
    scratch_types=[
        pltpu.VMEM((C,), jnp.int32),
        pltpu.VMEM((C,), jnp.int32),
        pltpu.VMEM((C, OUT), jnp.float32),
        pltpu.VMEM((C, OUT), jnp.float32),
        pltpu.VMEM((C,), jnp.int32),
        pltpu.VMEM((C,), jnp.int32),
        pltpu.VMEM((C, OUT), jnp.float32),
        pltpu.VMEM((C, OUT), jnp.float32),
        pltpu.VMEM_SHARED((NA, OUT), jnp.float32),
        pltpu.SemaphoreType.DMA,
        pltpu.SemaphoreType.DMA,
    ],
)


def _tc_final_body(msg_ref, rsd_ref, h_ref, scl_ref, off_ref, waggT_ref,
                   bagg_ref, wdstT_ref, bdst_ref, out_ref):
  acc = bagg_ref[...] + bdst_ref[...] + jnp.dot(
      h_ref[...], wdstT_ref[...], preferred_element_type=jnp.float32)
  msg = (msg_ref[0] + msg_ref[1]) * rsd_ref[...]
  waggT = waggT_ref[...]
  for hh in range(2):
    m = msg[:, hh * HD:(hh + 1) * HD]
    mean = jnp.mean(m, axis=1, keepdims=True)
    d = m - mean
    var = jnp.mean(d * d, axis=1, keepdims=True)
    hn = d * scl_ref[0, hh][None, :] * lax.rsqrt(var + 1e-9) \
        + off_ref[0, hh][None, :]
    acc = acc + jnp.dot(hn, waggT[hh * HD:(hh + 1) * HD, :],
                        preferred_element_type=jnp.float32)
  out_ref[...] = acc


def _final(msg, rsd, h, scale, offset, waggT, bagg, wdstT, bdst):
  return pl.pallas_call(
      _tc_final_body,
      grid=(N // RN,),
      in_specs=[
          pl.BlockSpec((2, RN, OUT), lambda i: (0, i, 0)),
          pl.BlockSpec((RN, OUT), lambda i: (i, 0)),
          pl.BlockSpec((RN, FH), lambda i: (i, 0)),
          pl.BlockSpec((1, 2, HD), lambda i: (0, 0, 0)),
          pl.BlockSpec((1, 2, HD), lambda i: (0, 0, 0)),
          pl.BlockSpec((OUT, OUT), lambda i: (0, 0)),
          pl.BlockSpec((1, OUT), lambda i: (0, 0)),
          pl.BlockSpec((FH, OUT), lambda i: (0, 0)),
          pl.BlockSpec((1, OUT), lambda i: (0, 0)),
      ],
      out_specs=pl.BlockSpec((RN, OUT), lambda i: (i, 0)),
      out_shape=jax.ShapeDtypeStruct((N, OUT), jnp.float32),
  )(msg, rsd, h, scale, offset, waggT, bagg, wdstT, bdst)


def kernel(x, edge_index, edge_attr, W_enc, b_enc, W_ee, b_ee, W_src, W_asrc,
           W_adst, W_aedge, scale, offset, W_agg, b_agg, W_dst, b_dst):
  pad_idx = N + jnp.arange(EP - E, dtype=jnp.int32) % (NA - N)
  src = jnp.concatenate([edge_index[0].astype(jnp.int32), pad_idx])
  dst = jnp.concatenate([edge_index[1].astype(jnp.int32), pad_idx])
  xp = jnp.zeros((NP, DF), jnp.float32).at[:N].set(x)
  eap = jnp.zeros((EP, edge_attr.shape[1]), jnp.float32).at[:E].set(edge_attr)

  h, f, asrc, adst = _node_proj(xp, W_enc.T, b_enc[None, :], W_src.T,
                                W_asrc.T, W_adst.T)
  ae = _edge_attn(eap, W_ee.T, b_ee[None, :], W_aedge.T)

  ex, sdst_p = _pass_a(src, dst, asrc, adst, ae)
  ssrc_p = _pass_a2(src, ex)
  rsd, g = _merge(sdst_p.reshape(2, NA, OUT), ssrc_p.reshape(2, NA, OUT), f)
  msg_p = _pass_b(src, dst, ex, g)

  return _final(msg_p.reshape(2, NA, OUT), rsd, h, scale, offset, W_agg.T,
                b_agg[None, :], W_dst.T, b_dst[None, :])
